# Initial kernel scaffold; baseline (speedup 1.0000x reference)
import jax
import jax.numpy as jnp
from jax.experimental import pallas as pl


def kernel(node_feats, edge_feats, edge_index, W_edge, b_edge, W_mlp, b_mlp):
    raise NotImplementedError("write your pallas kernel here")


# speedup vs baseline: 2.0950x; 2.0950x over previous
import functools

import jax
import jax.numpy as jnp
from jax import lax
from jax.experimental import pallas as pl
from jax.experimental.pallas import tpu as pltpu
from jax.experimental.pallas import tpu_sc as plsc

_N = 10000
_E = 320000
_D = 128
_H = 64
_EPS = 1e-07

_NS = 16
_KB = 32
_KC = 8
_B = _KC * _KB
_NB_BIG = 80
_NB_LAST = 50
_TILE_EDGES = _NB_BIG * _B
_RPT = 624
_LANES = 16


def _eh_body(x_ref, w_ref, b_ref, o_ref):
    o_ref[...] = (
        jnp.dot(x_ref[...], w_ref[0], preferred_element_type=jnp.float32)
        + b_ref[0]
    )


_BE = 3200


def _edge_encoder(edge_feats, W_split, b_split):
    nblk = _E // _BE
    return pl.pallas_call(
        _eh_body,
        grid=(nblk, 2),
        in_specs=[
            pl.BlockSpec((_BE, 16), lambda i, h: (i, 0)),
            pl.BlockSpec((1, 16, _H), lambda i, h: (h, 0, 0)),
            pl.BlockSpec((1, 1, _H), lambda i, h: (h, 0, 0)),
        ],
        out_specs=pl.BlockSpec((_BE, _H), lambda i, h: (h * nblk + i, 0)),
        out_shape=jax.ShapeDtypeStruct((2 * _E, _H), jnp.float32),
    )(edge_feats, W_split, b_split)


def _sc_body(node_hbm, eh_hbm, src_hbm, dst_hbm, out_hbm,
             srcv, dstv, gbuf, ebuf, acc_sh, sem):
    c = lax.axis_index("c")
    s = lax.axis_index("s")
    last = s == _NS - 1

    zero = jnp.zeros((_LANES,), jnp.float32)

    def _zrow(r, _):
        for k in range(_D // _LANES):
            gbuf[r, pl.ds(k * _LANES, _LANES)] = zero
        return 0
    lax.fori_loop(0, _B, _zrow, 0)

    r0 = pl.multiple_of(s * _RPT, 8)
    for off in range(0, _RPT, _B):
        nr = min(_B, _RPT - off)
        pltpu.sync_copy(gbuf.at[pl.ds(0, nr)], acc_sh.at[pl.ds(r0 + off, nr)])

    @pl.when(last)
    def _zero_tail():
        tail = _NS * _RPT
        pltpu.sync_copy(gbuf.at[pl.ds(0, _N - tail)],
                        acc_sh.at[pl.ds(tail, _N - tail)])

    plsc.subcore_barrier()

    nb = jnp.where(last, _NB_LAST, _NB_BIG)
    edge0 = s * _TILE_EDGES

    def _compute(col):
        def _row(r, _):
            for k in range(_H // _LANES):
                ofs = k * _LANES
                m = (jnp.maximum(gbuf[r, pl.ds(col + ofs, _LANES)]
                                 + ebuf[r, pl.ds(ofs, _LANES)], 0.0) + _EPS)
                w = jnp.exp(m)
                gbuf[r, pl.ds(ofs, _LANES)] = m * w
                gbuf[r, pl.ds(_H + ofs, _LANES)] = w
            return 0
        lax.fori_loop(0, _B, _row, 0)

    def _block(j, _):
        idx_row0 = pl.multiple_of((edge0 + j * _B) // _KB, 8)
        pltpu.sync_copy(src_hbm.at[pl.ds(idx_row0, _KC)], srcv)
        pltpu.sync_copy(dst_hbm.at[pl.ds(idx_row0, _KC)], dstv)

        ebase = pl.multiple_of(c * _E + edge0 + j * _B, 8)
        pltpu.sync_copy(eh_hbm.at[pl.ds(ebase, _B)], ebuf)
        cps = [
            pltpu.async_copy(node_hbm.at[srcv.at[t]],
                             gbuf.at[pl.ds(t * _KB, _KB)], sem)
            for t in range(_KC)
        ]
        for cp in cps:
            cp.wait()

        @pl.when(c == 0)
        def _c0():
            _compute(0)

        @pl.when(c == 1)
        def _c1():
            _compute(_H)

        for t in range(_KC):
            pltpu.sync_copy(gbuf.at[pl.ds(t * _KB, _KB)],
                            acc_sh.at[dstv.at[t]], add=True)
        return 0

    lax.fori_loop(0, nb, _block, 0)

    plsc.subcore_barrier()

    def _drain_chunk(row_base, nr):
        pltpu.sync_copy(acc_sh.at[pl.ds(row_base, nr)], gbuf.at[pl.ds(0, nr)])

        def _div(r, _):
            for k in range(_H // _LANES):
                ofs = k * _LANES
                num = gbuf[r, pl.ds(ofs, _LANES)]
                den = gbuf[r, pl.ds(_H + ofs, _LANES)]
                ebuf[r, pl.ds(ofs, _LANES)] = num / jnp.maximum(den, 1e-16)
            return 0
        lax.fori_loop(0, nr, _div, 0)
        pltpu.sync_copy(ebuf.at[pl.ds(0, nr)],
                        out_hbm.at[pl.ds(c * _N + row_base, nr)])

    def _drain(row_base, total):
        for off in range(0, total, _B):
            _drain_chunk(row_base + off, min(_B, total - off))

    _drain(r0, _RPT)

    @pl.when(last)
    def _drain_tail():
        _drain(_NS * _RPT, _N - _NS * _RPT)


def _sc_aggregate(node_feats, eh_cat, src2d, dst2d):
    mesh = plsc.VectorSubcoreMesh(core_axis_name="c", subcore_axis_name="s")
    kern = functools.partial(
        pl.kernel,
        mesh=mesh,
        compiler_params=pltpu.CompilerParams(use_tc_tiling_on_sc=False),
        out_type=jax.ShapeDtypeStruct((2 * _N, _H), jnp.float32),
        scratch_types=[
            pltpu.VMEM((_KC, _KB), jnp.int32),
            pltpu.VMEM((_KC, _KB), jnp.int32),
            pltpu.VMEM((_B, _D), jnp.float32),
            pltpu.VMEM((_B, _H), jnp.float32),
            pltpu.VMEM_SHARED((_N, _D), jnp.float32),
            pltpu.SemaphoreType.DMA,
        ],
    )(_sc_body)
    return kern(node_feats, eh_cat, src2d, dst2d)


def _mlp_body(x_ref, lo_ref, hi_ref, w_ref, b_ref, o_ref):
    feats = x_ref[...] + jnp.concatenate([lo_ref[...], hi_ref[...]], axis=1)
    o_ref[...] = (
        jnp.dot(feats, w_ref[...], preferred_element_type=jnp.float32)
        + b_ref[...]
    )


_BN = 2000


def _output_mlp(node_feats, agg_cat, W_mlp, b_mlp2d):
    nblk = _N // _BN
    return pl.pallas_call(
        _mlp_body,
        grid=(nblk,),
        in_specs=[
            pl.BlockSpec((_BN, _D), lambda i: (i, 0)),
            pl.BlockSpec((_BN, _H), lambda i: (i, 0)),
            pl.BlockSpec((_BN, _H), lambda i: (nblk + i, 0)),
            pl.BlockSpec((_D, _D), lambda i: (0, 0)),
            pl.BlockSpec((1, _D), lambda i: (0, 0)),
        ],
        out_specs=pl.BlockSpec((_BN, _D), lambda i: (i, 0)),
        out_shape=jax.ShapeDtypeStruct((_N, _D), jnp.float32),
    )(node_feats, agg_cat, agg_cat, W_mlp, b_mlp2d)


def kernel(node_feats, edge_feats, edge_index, W_edge, b_edge, W_mlp, b_mlp):
    src2d = edge_index[0].reshape(_E // _KB, _KB)
    dst2d = edge_index[1].reshape(_E // _KB, _KB)
    W_split = W_edge.reshape(16, 2, _H).transpose(1, 0, 2)
    b_split = b_edge.reshape(2, 1, _H)
    eh_cat = _edge_encoder(edge_feats, W_split, b_split)
    agg_cat = _sc_aggregate(node_feats, eh_cat, src2d, dst2d)
    return _output_mlp(node_feats, agg_cat, W_mlp, b_mlp.reshape(1, _D))


# speedup vs baseline: 2.5589x; 1.2214x over previous
import functools

import jax
import jax.numpy as jnp
from jax import lax
from jax.experimental import pallas as pl
from jax.experimental.pallas import tpu as pltpu
from jax.experimental.pallas import tpu_sc as plsc

_N = 10000
_E = 320000
_D = 128
_H = 64
_EPS = 1e-07

_NS = 16
_B = 80
_NB = _E // _NS // _B
_TILE_EDGES = _NB * _B
_RPT = _N // _NS
_LANES = 16


def _eh_body(x_ref, w_ref, b_ref, o_ref):
    o_ref[...] = (
        jnp.dot(x_ref[...], w_ref[0], preferred_element_type=jnp.float32)
        + b_ref[0]
    )


_BE = 3200


def _edge_encoder(edge_feats, W_split, b_split):
    nblk = _E // _BE
    return pl.pallas_call(
        _eh_body,
        grid=(nblk, 2),
        in_specs=[
            pl.BlockSpec((_BE, 16), lambda i, h: (i, 0)),
            pl.BlockSpec((1, 16, _H), lambda i, h: (h, 0, 0)),
            pl.BlockSpec((1, 1, _H), lambda i, h: (h, 0, 0)),
        ],
        out_specs=pl.BlockSpec((_BE, _H), lambda i, h: (h * nblk + i, 0)),
        out_shape=jax.ShapeDtypeStruct((2 * _E, _H), jnp.float32),
    )(edge_feats, W_split, b_split)


def _sc_body(node_hbm, eh_hbm, idx_hbm, out_hbm,
             idxv, gbuf, ebuf, acc_sh, lsem, isem, ssem):
    c = lax.axis_index("c")
    s = lax.axis_index("s")
    last = s == _NS - 1

    zero = jnp.zeros((_LANES,), jnp.float32)

    def _zrow(r, _):
        for k in range(_D // _LANES):
            gbuf[0, r, pl.ds(k * _LANES, _LANES)] = zero
        return 0
    lax.fori_loop(0, _B, _zrow, 0)

    r0 = s * _RPT
    for off in range(0, _RPT, _B):
        nr = min(_B, _RPT - off)
        pltpu.sync_copy(gbuf.at[0, pl.ds(0, nr)],
                        acc_sh.at[pl.ds(r0 + off, nr)])

    plsc.subcore_barrier()

    idx_row0 = s * _NB
    edge0 = s * _TILE_EDGES

    def _load_idx(j, start):
        cp = pltpu.make_async_copy(idx_hbm.at[pl.ds(idx_row0 + j, 1)],
                                   idxv.at[lax.rem(j, 4)], isem)
        if start:
            cp.start()
        else:
            cp.wait()

    def _load_data(j, start):
        p = lax.rem(j, 3)
        q = lax.rem(j, 4)
        g = pltpu.make_async_copy(node_hbm.at[idxv.at[q, 0, 0]],
                                  gbuf.at[p], lsem)
        e = pltpu.make_async_copy(eh_hbm.at[pl.ds(c * _E + edge0 + j * _B, _B)],
                                  ebuf.at[p], lsem)
        if start:
            g.start()
            e.start()
        else:
            g.wait()
            e.wait()

    def _scatter(j, start):
        p = lax.rem(j, 3)
        q = lax.rem(j, 4)
        cp = pltpu.make_async_copy(gbuf.at[p], acc_sh.at[idxv.at[q, 0, 1]],
                                   ssem.at[p])
        if start:
            cp.start(add=True)
        else:
            cp.wait()

    def _compute(j):
        p = lax.rem(j, 3)

        def _half(col):
            def _row(r, _):
                for k in range(_H // _LANES):
                    ofs = k * _LANES
                    m = (jnp.maximum(gbuf[p, r, pl.ds(col + ofs, _LANES)]
                                     + ebuf[p, r, pl.ds(ofs, _LANES)], 0.0)
                         + _EPS)
                    w = jnp.exp(m)
                    gbuf[p, r, pl.ds(ofs, _LANES)] = m * w
                    gbuf[p, r, pl.ds(_H + ofs, _LANES)] = w
                return 0
            lax.fori_loop(0, _B, _row, 0)

        @pl.when(c == 0)
        def _c0():
            _half(0)

        @pl.when(c == 1)
        def _c1():
            _half(_H)

    pltpu.sync_copy(idx_hbm.at[pl.ds(idx_row0, 1)], idxv.at[0])
    _load_data(0, True)
    _load_idx(1, True)

    def _iter(j, _):
        _load_data(j, False)

        @pl.when(j >= 2)
        def _():
            _scatter(j - 2, False)

        @pl.when(j + 1 <= _NB - 1)
        def _():
            _load_idx(j + 1, False)
            _load_data(j + 1, True)

        @pl.when(j + 2 <= _NB - 1)
        def _():
            _load_idx(j + 2, True)

        _compute(j)
        _scatter(j, True)
        return 0

    lax.fori_loop(0, _NB, _iter, 0)
    _scatter(_NB - 2, False)
    _scatter(_NB - 1, False)

    plsc.subcore_barrier()

    def _drain_chunk(row_base, nr):
        pltpu.sync_copy(acc_sh.at[pl.ds(row_base, nr)],
                        gbuf.at[0, pl.ds(0, nr)])

        def _div(r, _):
            for k in range(_H // _LANES):
                ofs = k * _LANES
                num = gbuf[0, r, pl.ds(ofs, _LANES)]
                den = gbuf[0, r, pl.ds(_H + ofs, _LANES)]
                ebuf[0, r, pl.ds(ofs, _LANES)] = num / jnp.maximum(den, 1e-16)
            return 0
        lax.fori_loop(0, nr, _div, 0)
        pltpu.sync_copy(ebuf.at[0, pl.ds(0, nr)],
                        out_hbm.at[pl.ds(c * _N + row_base, nr)])

    for off in range(0, _RPT, _B):
        _drain_chunk(r0 + off, min(_B, _RPT - off))


def _sc_aggregate(node_feats, eh_cat, idx2):
    mesh = plsc.VectorSubcoreMesh(core_axis_name="c", subcore_axis_name="s")
    kern = functools.partial(
        pl.kernel,
        mesh=mesh,
        compiler_params=pltpu.CompilerParams(use_tc_tiling_on_sc=False),
        out_type=jax.ShapeDtypeStruct((2 * _N, _H), jnp.float32),
        scratch_types=[
            pltpu.VMEM((4, 1, 2, _B), jnp.int32),
            pltpu.VMEM((3, _B, _D), jnp.float32),
            pltpu.VMEM((3, _B, _H), jnp.float32),
            pltpu.VMEM_SHARED((_N, _D), jnp.float32),
            pltpu.SemaphoreType.DMA,
            pltpu.SemaphoreType.DMA,
            pltpu.SemaphoreType.DMA((3,)),
        ],
    )(_sc_body)
    return kern(node_feats, eh_cat, idx2)


def _mlp_body(x_ref, lo_ref, hi_ref, w_ref, b_ref, o_ref):
    feats = x_ref[...] + jnp.concatenate([lo_ref[...], hi_ref[...]], axis=1)
    o_ref[...] = (
        jnp.dot(feats, w_ref[...], preferred_element_type=jnp.float32)
        + b_ref[...]
    )


_BN = 2000


def _output_mlp(node_feats, agg_cat, W_mlp, b_mlp2d):
    nblk = _N // _BN
    return pl.pallas_call(
        _mlp_body,
        grid=(nblk,),
        in_specs=[
            pl.BlockSpec((_BN, _D), lambda i: (i, 0)),
            pl.BlockSpec((_BN, _H), lambda i: (i, 0)),
            pl.BlockSpec((_BN, _H), lambda i: (nblk + i, 0)),
            pl.BlockSpec((_D, _D), lambda i: (0, 0)),
            pl.BlockSpec((1, _D), lambda i: (0, 0)),
        ],
        out_specs=pl.BlockSpec((_BN, _D), lambda i: (i, 0)),
        out_shape=jax.ShapeDtypeStruct((_N, _D), jnp.float32),
    )(node_feats, agg_cat, agg_cat, W_mlp, b_mlp2d)


def kernel(node_feats, edge_feats, edge_index, W_edge, b_edge, W_mlp, b_mlp):
    idx2 = jnp.stack(
        [edge_index[0].reshape(_E // _B, _B),
         edge_index[1].reshape(_E // _B, _B)], axis=1)
    W_split = W_edge.reshape(16, 2, _H).transpose(1, 0, 2)
    b_split = b_edge.reshape(2, 1, _H)
    eh_cat = _edge_encoder(edge_feats, W_split, b_split)
    agg_cat = _sc_aggregate(node_feats, eh_cat, idx2)
    return _output_mlp(node_feats, agg_cat, W_mlp, b_mlp.reshape(1, _D))


# speedup vs baseline: 2.6309x; 1.0281x over previous
import functools

import jax
import jax.numpy as jnp
from jax import lax
from jax.experimental import pallas as pl
from jax.experimental.pallas import tpu as pltpu
from jax.experimental.pallas import tpu_sc as plsc

_N = 10000
_E = 320000
_D = 128
_H = 64
_EPS = 1e-07

_NS = 16
_B = 80
_NB = _E // _NS // _B
_TILE_EDGES = _NB * _B
_RPT = _N // _NS
_LANES = 16


def _eh_body(x_ref, w_ref, b_ref, o_ref):
    o_ref[...] = (
        jnp.dot(x_ref[...], w_ref[0], preferred_element_type=jnp.float32)
        + b_ref[0]
    )


_BE = 3200


def _edge_encoder(edge_feats, W_split, b_split):
    nblk = _E // _BE
    return pl.pallas_call(
        _eh_body,
        grid=(nblk, 2),
        in_specs=[
            pl.BlockSpec((_BE, 16), lambda i, h: (i, 0)),
            pl.BlockSpec((1, 16, _H), lambda i, h: (h, 0, 0)),
            pl.BlockSpec((1, 1, _H), lambda i, h: (h, 0, 0)),
        ],
        out_specs=pl.BlockSpec((_BE, _H), lambda i, h: (h * nblk + i, 0)),
        out_shape=jax.ShapeDtypeStruct((2 * _E, _H), jnp.float32),
    )(edge_feats, W_split, b_split)


def _sc_body(node_hbm, eh_hbm, idx_hbm, out_hbm,
             idxv, gbuf, ebuf, acc_sh, lsem, isem, ssem):
    c = lax.axis_index("c")
    s = lax.axis_index("s")
    last = s == _NS - 1

    zero = jnp.zeros((_LANES,), jnp.float32)

    def _zrow(r, _):
        for k in range(_D // _LANES):
            gbuf[0, r, pl.ds(k * _LANES, _LANES)] = zero
        return 0
    lax.fori_loop(0, _B, _zrow, 0)

    r0 = s * _RPT
    for off in range(0, _RPT, _B):
        nr = min(_B, _RPT - off)
        pltpu.sync_copy(gbuf.at[0, pl.ds(0, nr)],
                        acc_sh.at[pl.ds(r0 + off, nr)])

    plsc.subcore_barrier()

    idx_row0 = s * _NB
    edge0 = s * _TILE_EDGES

    def _load_idx(j, start):
        cp = pltpu.make_async_copy(idx_hbm.at[pl.ds(idx_row0 + j, 1)],
                                   idxv.at[lax.rem(j, 4)], isem)
        if start:
            cp.start()
        else:
            cp.wait()

    def _load_data(j, start):
        p = lax.rem(j, 3)
        q = lax.rem(j, 4)
        g = pltpu.make_async_copy(node_hbm.at[idxv.at[q, 0, 0]],
                                  gbuf.at[p], lsem)
        e = pltpu.make_async_copy(eh_hbm.at[pl.ds(c * _E + edge0 + j * _B, _B)],
                                  ebuf.at[p], lsem)
        if start:
            g.start()
            e.start()
        else:
            g.wait()
            e.wait()

    def _scatter(j, start):
        p = lax.rem(j, 3)
        q = lax.rem(j, 4)
        cp = pltpu.make_async_copy(gbuf.at[p], acc_sh.at[idxv.at[q, 0, 1]],
                                   ssem.at[p])
        if start:
            cp.start(add=True)
        else:
            cp.wait()

    def _compute(j):
        p = lax.rem(j, 3)

        def _half(col):
            def _rows(i, _):
                base = i * 4
                for u in range(4):
                    r = base + u
                    for k in range(_H // _LANES):
                        ofs = k * _LANES
                        m = (jnp.maximum(gbuf[p, r, pl.ds(col + ofs, _LANES)]
                                         + ebuf[p, r, pl.ds(ofs, _LANES)], 0.0)
                             + _EPS)
                        w = jnp.exp(m)
                        gbuf[p, r, pl.ds(ofs, _LANES)] = m * w
                        gbuf[p, r, pl.ds(_H + ofs, _LANES)] = w
                return 0
            lax.fori_loop(0, _B // 4, _rows, 0)

        @pl.when(c == 0)
        def _c0():
            _half(0)

        @pl.when(c == 1)
        def _c1():
            _half(_H)

    pltpu.sync_copy(idx_hbm.at[pl.ds(idx_row0, 1)], idxv.at[0])
    _load_data(0, True)
    _load_idx(1, True)

    def _iter(j, _):
        _load_data(j, False)

        @pl.when(j >= 2)
        def _():
            _scatter(j - 2, False)

        @pl.when(j + 1 <= _NB - 1)
        def _():
            _load_idx(j + 1, False)
            _load_data(j + 1, True)

        @pl.when(j + 2 <= _NB - 1)
        def _():
            _load_idx(j + 2, True)

        _compute(j)
        _scatter(j, True)
        return 0

    lax.fori_loop(0, _NB, _iter, 0)
    _scatter(_NB - 2, False)
    _scatter(_NB - 1, False)

    plsc.subcore_barrier()

    def _drain_chunk(row_base, nr):
        pltpu.sync_copy(acc_sh.at[pl.ds(row_base, nr)],
                        gbuf.at[0, pl.ds(0, nr)])

        def _div(r, _):
            for k in range(_H // _LANES):
                ofs = k * _LANES
                num = gbuf[0, r, pl.ds(ofs, _LANES)]
                den = gbuf[0, r, pl.ds(_H + ofs, _LANES)]
                ebuf[0, r, pl.ds(ofs, _LANES)] = num / jnp.maximum(den, 1e-16)
            return 0
        lax.fori_loop(0, nr, _div, 0)
        pltpu.sync_copy(ebuf.at[0, pl.ds(0, nr)],
                        out_hbm.at[pl.ds(c * _N + row_base, nr)])

    for off in range(0, _RPT, _B):
        _drain_chunk(r0 + off, min(_B, _RPT - off))


def _sc_aggregate(node_feats, eh_cat, idx2):
    mesh = plsc.VectorSubcoreMesh(core_axis_name="c", subcore_axis_name="s")
    kern = functools.partial(
        pl.kernel,
        mesh=mesh,
        compiler_params=pltpu.CompilerParams(use_tc_tiling_on_sc=False),
        out_type=jax.ShapeDtypeStruct((2 * _N, _H), jnp.float32),
        scratch_types=[
            pltpu.VMEM((4, 1, 2, _B), jnp.int32),
            pltpu.VMEM((3, _B, _D), jnp.float32),
            pltpu.VMEM((3, _B, _H), jnp.float32),
            pltpu.VMEM_SHARED((_N, _D), jnp.float32),
            pltpu.SemaphoreType.DMA,
            pltpu.SemaphoreType.DMA,
            pltpu.SemaphoreType.DMA((3,)),
        ],
    )(_sc_body)
    return kern(node_feats, eh_cat, idx2)


def _mlp_body(x_ref, lo_ref, hi_ref, w_ref, b_ref, o_ref):
    feats = x_ref[...] + jnp.concatenate([lo_ref[...], hi_ref[...]], axis=1)
    o_ref[...] = (
        jnp.dot(feats, w_ref[...], preferred_element_type=jnp.float32)
        + b_ref[...]
    )


_BN = 2000


def _output_mlp(node_feats, agg_cat, W_mlp, b_mlp2d):
    nblk = _N // _BN
    return pl.pallas_call(
        _mlp_body,
        grid=(nblk,),
        in_specs=[
            pl.BlockSpec((_BN, _D), lambda i: (i, 0)),
            pl.BlockSpec((_BN, _H), lambda i: (i, 0)),
            pl.BlockSpec((_BN, _H), lambda i: (nblk + i, 0)),
            pl.BlockSpec((_D, _D), lambda i: (0, 0)),
            pl.BlockSpec((1, _D), lambda i: (0, 0)),
        ],
        out_specs=pl.BlockSpec((_BN, _D), lambda i: (i, 0)),
        out_shape=jax.ShapeDtypeStruct((_N, _D), jnp.float32),
    )(node_feats, agg_cat, agg_cat, W_mlp, b_mlp2d)


def kernel(node_feats, edge_feats, edge_index, W_edge, b_edge, W_mlp, b_mlp):
    idx2 = jnp.stack(
        [edge_index[0].reshape(_E // _B, _B),
         edge_index[1].reshape(_E // _B, _B)], axis=1)
    W_split = W_edge.reshape(16, 2, _H).transpose(1, 0, 2)
    b_split = b_edge.reshape(2, 1, _H)
    eh_cat = _edge_encoder(edge_feats, W_split, b_split)
    agg_cat = _sc_aggregate(node_feats, eh_cat, idx2)
    return _output_mlp(node_feats, agg_cat, W_mlp, b_mlp.reshape(1, _D))


# speedup vs baseline: 4.7394x; 1.8014x over previous
import functools

import jax
import jax.numpy as jnp
from jax import lax
from jax.experimental import pallas as pl
from jax.experimental.pallas import tpu as pltpu
from jax.experimental.pallas import tpu_sc as plsc

_N = 10000
_E = 320000
_D = 128
_H = 64
_EPS = 1e-07

_NS = 16
_B = 80
_NB = _E // _NS // _B
_TILE_EDGES = _NB * _B
_RPT = _N // _NS
_LANES = 16


def _eh_body(x_ref, w_ref, b_ref, o_ref):
    o_ref[...] = (
        jnp.dot(x_ref[...], w_ref[0], preferred_element_type=jnp.float32)
        + b_ref[0]
    )


_BE = 3200


def _edge_encoder(edge_feats, W_split, b_split):
    nblk = _E // _BE
    return pl.pallas_call(
        _eh_body,
        grid=(nblk, 2),
        in_specs=[
            pl.BlockSpec((_BE, 16), lambda i, h: (i, 0)),
            pl.BlockSpec((1, 16, _H), lambda i, h: (h, 0, 0)),
            pl.BlockSpec((1, 1, _H), lambda i, h: (h, 0, 0)),
        ],
        out_specs=pl.BlockSpec((_BE, _H), lambda i, h: (h * nblk + i, 0)),
        out_shape=jax.ShapeDtypeStruct((2 * _E, _H), jnp.float32),
    )(edge_feats, W_split, b_split)


def _sc_body(node_hbm, eh_hbm, idx_hbm, out_hbm,
             idxv, gbuf, ebuf, acc_sh, lsem, isem, ssem):
    c = lax.axis_index("c")
    s = lax.axis_index("s")
    last = s == _NS - 1

    zero = jnp.zeros((_LANES,), jnp.float32)

    @plsc.parallel_loop(0, _B, unroll=4)
    def _zrow(r):
        for k in range(_D // _LANES):
            gbuf[0, r, pl.ds(k * _LANES, _LANES)] = zero

    r0 = s * _RPT
    for off in range(0, _RPT, _B):
        nr = min(_B, _RPT - off)
        pltpu.sync_copy(gbuf.at[0, pl.ds(0, nr)],
                        acc_sh.at[pl.ds(r0 + off, nr)])

    plsc.subcore_barrier()

    idx_row0 = s * _NB
    edge0 = s * _TILE_EDGES

    def _load_idx(j, start):
        cp = pltpu.make_async_copy(idx_hbm.at[pl.ds(idx_row0 + j, 1)],
                                   idxv.at[lax.rem(j, 4)], isem)
        if start:
            cp.start()
        else:
            cp.wait()

    def _load_data(j, start):
        p = lax.rem(j, 3)
        q = lax.rem(j, 4)
        g = pltpu.make_async_copy(node_hbm.at[idxv.at[q, 0, 0]],
                                  gbuf.at[p], lsem)
        e = pltpu.make_async_copy(eh_hbm.at[pl.ds(c * _E + edge0 + j * _B, _B)],
                                  ebuf.at[p], lsem)
        if start:
            g.start()
            e.start()
        else:
            g.wait()
            e.wait()

    def _scatter(j, start):
        p = lax.rem(j, 3)
        q = lax.rem(j, 4)
        cp = pltpu.make_async_copy(gbuf.at[p], acc_sh.at[idxv.at[q, 0, 1]],
                                   ssem.at[p])
        if start:
            cp.start(add=True)
        else:
            cp.wait()

    def _compute(j):
        p = lax.rem(j, 3)

        def _half(col):
            @plsc.parallel_loop(0, _B, unroll=4)
            def _row(r):
                for k in range(_H // _LANES):
                    ofs = k * _LANES
                    m = (jnp.maximum(gbuf[p, r, pl.ds(col + ofs, _LANES)]
                                     + ebuf[p, r, pl.ds(ofs, _LANES)], 0.0)
                         + _EPS)
                    w = jnp.exp(m)
                    gbuf[p, r, pl.ds(ofs, _LANES)] = m * w
                    gbuf[p, r, pl.ds(_H + ofs, _LANES)] = w

        @pl.when(c == 0)
        def _c0():
            _half(0)

        @pl.when(c == 1)
        def _c1():
            _half(_H)

    pltpu.sync_copy(idx_hbm.at[pl.ds(idx_row0, 1)], idxv.at[0])
    _load_data(0, True)
    _load_idx(1, True)

    def _iter(j, _):
        _load_data(j, False)

        @pl.when(j >= 2)
        def _():
            _scatter(j - 2, False)

        @pl.when(j + 1 <= _NB - 1)
        def _():
            _load_idx(j + 1, False)
            _load_data(j + 1, True)

        @pl.when(j + 2 <= _NB - 1)
        def _():
            _load_idx(j + 2, True)

        _compute(j)
        _scatter(j, True)
        return 0

    lax.fori_loop(0, _NB, _iter, 0)
    _scatter(_NB - 2, False)
    _scatter(_NB - 1, False)

    plsc.subcore_barrier()

    def _drain_chunk(row_base, nr):
        pltpu.sync_copy(acc_sh.at[pl.ds(row_base, nr)],
                        gbuf.at[0, pl.ds(0, nr)])

        @plsc.parallel_loop(0, nr, unroll=4)
        def _div(r):
            for k in range(_H // _LANES):
                ofs = k * _LANES
                num = gbuf[0, r, pl.ds(ofs, _LANES)]
                den = gbuf[0, r, pl.ds(_H + ofs, _LANES)]
                ebuf[0, r, pl.ds(ofs, _LANES)] = num / jnp.maximum(den, 1e-16)
        pltpu.sync_copy(ebuf.at[0, pl.ds(0, nr)],
                        out_hbm.at[pl.ds(c * _N + row_base, nr)])

    for off in range(0, _RPT, _B):
        _drain_chunk(r0 + off, min(_B, _RPT - off))


def _sc_aggregate(node_feats, eh_cat, idx2):
    mesh = plsc.VectorSubcoreMesh(core_axis_name="c", subcore_axis_name="s")
    kern = functools.partial(
        pl.kernel,
        mesh=mesh,
        compiler_params=pltpu.CompilerParams(use_tc_tiling_on_sc=False),
        out_type=jax.ShapeDtypeStruct((2 * _N, _H), jnp.float32),
        scratch_types=[
            pltpu.VMEM((4, 1, 2, _B), jnp.int32),
            pltpu.VMEM((3, _B, _D), jnp.float32),
            pltpu.VMEM((3, _B, _H), jnp.float32),
            pltpu.VMEM_SHARED((_N, _D), jnp.float32),
            pltpu.SemaphoreType.DMA,
            pltpu.SemaphoreType.DMA,
            pltpu.SemaphoreType.DMA((3,)),
        ],
    )(_sc_body)
    return kern(node_feats, eh_cat, idx2)


def _mlp_body(x_ref, lo_ref, hi_ref, w_ref, b_ref, o_ref):
    feats = x_ref[...] + jnp.concatenate([lo_ref[...], hi_ref[...]], axis=1)
    o_ref[...] = (
        jnp.dot(feats, w_ref[...], preferred_element_type=jnp.float32)
        + b_ref[...]
    )


_BN = 2000


def _output_mlp(node_feats, agg_cat, W_mlp, b_mlp2d):
    nblk = _N // _BN
    return pl.pallas_call(
        _mlp_body,
        grid=(nblk,),
        in_specs=[
            pl.BlockSpec((_BN, _D), lambda i: (i, 0)),
            pl.BlockSpec((_BN, _H), lambda i: (i, 0)),
            pl.BlockSpec((_BN, _H), lambda i: (nblk + i, 0)),
            pl.BlockSpec((_D, _D), lambda i: (0, 0)),
            pl.BlockSpec((1, _D), lambda i: (0, 0)),
        ],
        out_specs=pl.BlockSpec((_BN, _D), lambda i: (i, 0)),
        out_shape=jax.ShapeDtypeStruct((_N, _D), jnp.float32),
    )(node_feats, agg_cat, agg_cat, W_mlp, b_mlp2d)


def kernel(node_feats, edge_feats, edge_index, W_edge, b_edge, W_mlp, b_mlp):
    idx2 = jnp.stack(
        [edge_index[0].reshape(_E // _B, _B),
         edge_index[1].reshape(_E // _B, _B)], axis=1)
    W_split = W_edge.reshape(16, 2, _H).transpose(1, 0, 2)
    b_split = b_edge.reshape(2, 1, _H)
    eh_cat = _edge_encoder(edge_feats, W_split, b_split)
    agg_cat = _sc_aggregate(node_feats, eh_cat, idx2)
    return _output_mlp(node_feats, agg_cat, W_mlp, b_mlp.reshape(1, _D))


# speedup vs baseline: 8.6096x; 1.8166x over previous
import functools

import jax
import jax.numpy as jnp
from jax import lax
from jax.experimental import pallas as pl
from jax.experimental.pallas import tpu as pltpu
from jax.experimental.pallas import tpu_sc as plsc

_N = 10000
_E = 320000
_D = 128
_H = 64
_EPS = 1e-07

_NS = 16
_B = 80
_NB = _E // _NS // _B
_TILE_EDGES = _NB * _B
_RPT = _N // _NS
_LANES = 16


def _eh_body(x_ref, w_ref, b_ref, o_ref):
    o_ref[...] = (
        jnp.dot(x_ref[...], w_ref[...], preferred_element_type=jnp.float32)
        + b_ref[...]
    )


_BE = 8000


def _edge_encoder(edge_feats, W_edge, b_edge2d):
    return pl.pallas_call(
        _eh_body,
        grid=(_E // _BE,),
        in_specs=[
            pl.BlockSpec((_BE, 16), lambda i: (i, 0)),
            pl.BlockSpec((16, _D), lambda i: (0, 0)),
            pl.BlockSpec((1, _D), lambda i: (0, 0)),
        ],
        out_specs=pl.BlockSpec((_BE, _D), lambda i: (i, 0)),
        out_shape=jax.ShapeDtypeStruct((_E, _D), jnp.float32),
    )(edge_feats, W_edge, b_edge2d)


def _sc_body(node_hbm, eh_hbm, idx_hbm, out_hbm,
             idxv, gbuf, ebuf, acc_sh, lsem, isem, ssem):
    c = lax.axis_index("c")
    s = lax.axis_index("s")
    last = s == _NS - 1

    zero = jnp.zeros((_LANES,), jnp.float32)

    @plsc.parallel_loop(0, _B, unroll=4)
    def _zrow(r):
        for k in range(_D // _LANES):
            gbuf[0, r, pl.ds(k * _LANES, _LANES)] = zero

    r0 = s * _RPT
    for off in range(0, _RPT, _B):
        nr = min(_B, _RPT - off)
        pltpu.sync_copy(gbuf.at[0, pl.ds(0, nr)],
                        acc_sh.at[pl.ds(r0 + off, nr)])

    plsc.subcore_barrier()

    edge0 = s * _TILE_EDGES

    def _load_idx(j, start):
        q = lax.rem(j, 4)
        e0 = edge0 + j * _B
        cps = [
            pltpu.make_async_copy(idx_hbm.at[0, pl.ds(e0, _B)],
                                  idxv.at[q, 0], isem),
            pltpu.make_async_copy(idx_hbm.at[1, pl.ds(e0, _B)],
                                  idxv.at[q, 1], isem),
        ]
        for cp in cps:
            if start:
                cp.start()
            else:
                cp.wait()

    def _load_data(j, start):
        p = lax.rem(j, 3)
        q = lax.rem(j, 4)
        g = pltpu.make_async_copy(node_hbm.at[idxv.at[q, 0]],
                                  gbuf.at[p], lsem)
        e = pltpu.make_async_copy(
            eh_hbm.at[pl.ds(edge0 + j * _B, _B), pl.ds(c * _H, _H)],
            ebuf.at[p], lsem)
        if start:
            g.start()
            e.start()
        else:
            g.wait()
            e.wait()

    def _scatter(j, start):
        p = lax.rem(j, 3)
        q = lax.rem(j, 4)
        cp = pltpu.make_async_copy(gbuf.at[p], acc_sh.at[idxv.at[q, 1]],
                                   ssem.at[p])
        if start:
            cp.start(add=True)
        else:
            cp.wait()

    def _compute(j):
        p = lax.rem(j, 3)

        def _half(col):
            @plsc.parallel_loop(0, _B, unroll=4)
            def _row(r):
                for k in range(_H // _LANES):
                    ofs = k * _LANES
                    m = (jnp.maximum(gbuf[p, r, pl.ds(col + ofs, _LANES)]
                                     + ebuf[p, r, pl.ds(ofs, _LANES)], 0.0)
                         + _EPS)
                    w = jnp.exp(m)
                    gbuf[p, r, pl.ds(ofs, _LANES)] = m * w
                    gbuf[p, r, pl.ds(_H + ofs, _LANES)] = w

        @pl.when(c == 0)
        def _c0():
            _half(0)

        @pl.when(c == 1)
        def _c1():
            _half(_H)

    _load_idx(0, True)
    _load_idx(0, False)
    _load_data(0, True)
    _load_idx(1, True)

    def _iter(j, _):
        _load_data(j, False)

        @pl.when(j >= 2)
        def _():
            _scatter(j - 2, False)

        @pl.when(j + 1 <= _NB - 1)
        def _():
            _load_idx(j + 1, False)
            _load_data(j + 1, True)

        @pl.when(j + 2 <= _NB - 1)
        def _():
            _load_idx(j + 2, True)

        _compute(j)
        _scatter(j, True)
        return 0

    lax.fori_loop(0, _NB, _iter, 0)
    _scatter(_NB - 2, False)
    _scatter(_NB - 1, False)

    plsc.subcore_barrier()

    def _drain_chunk(row_base, nr):
        pltpu.sync_copy(acc_sh.at[pl.ds(row_base, nr)],
                        gbuf.at[0, pl.ds(0, nr)])

        @plsc.parallel_loop(0, nr, unroll=4)
        def _div(r):
            for k in range(_H // _LANES):
                ofs = k * _LANES
                num = gbuf[0, r, pl.ds(ofs, _LANES)]
                den = gbuf[0, r, pl.ds(_H + ofs, _LANES)]
                ebuf[0, r, pl.ds(ofs, _LANES)] = num / jnp.maximum(den, 1e-16)
        pltpu.sync_copy(ebuf.at[0, pl.ds(0, nr)],
                        out_hbm.at[pl.ds(c * _N + row_base, nr)])

    for off in range(0, _RPT, _B):
        _drain_chunk(r0 + off, min(_B, _RPT - off))


def _sc_aggregate(node_feats, eh_cat, edge_index):
    mesh = plsc.VectorSubcoreMesh(core_axis_name="c", subcore_axis_name="s")
    kern = functools.partial(
        pl.kernel,
        mesh=mesh,
        compiler_params=pltpu.CompilerParams(use_tc_tiling_on_sc=False),
        out_type=jax.ShapeDtypeStruct((2 * _N, _H), jnp.float32),
        scratch_types=[
            pltpu.VMEM((4, 2, _B), jnp.int32),
            pltpu.VMEM((3, _B, _D), jnp.float32),
            pltpu.VMEM((3, _B, _H), jnp.float32),
            pltpu.VMEM_SHARED((_N, _D), jnp.float32),
            pltpu.SemaphoreType.DMA,
            pltpu.SemaphoreType.DMA,
            pltpu.SemaphoreType.DMA((3,)),
        ],
    )(_sc_body)
    return kern(node_feats, eh_cat, edge_index)


def _mlp_body(x_ref, lo_ref, hi_ref, w_ref, b_ref, o_ref):
    feats = x_ref[...] + jnp.concatenate([lo_ref[...], hi_ref[...]], axis=1)
    o_ref[...] = (
        jnp.dot(feats, w_ref[...], preferred_element_type=jnp.float32)
        + b_ref[...]
    )


_BN = 2000


def _output_mlp(node_feats, agg_cat, W_mlp, b_mlp2d):
    nblk = _N // _BN
    return pl.pallas_call(
        _mlp_body,
        grid=(nblk,),
        in_specs=[
            pl.BlockSpec((_BN, _D), lambda i: (i, 0)),
            pl.BlockSpec((_BN, _H), lambda i: (i, 0)),
            pl.BlockSpec((_BN, _H), lambda i: (nblk + i, 0)),
            pl.BlockSpec((_D, _D), lambda i: (0, 0)),
            pl.BlockSpec((1, _D), lambda i: (0, 0)),
        ],
        out_specs=pl.BlockSpec((_BN, _D), lambda i: (i, 0)),
        out_shape=jax.ShapeDtypeStruct((_N, _D), jnp.float32),
    )(node_feats, agg_cat, agg_cat, W_mlp, b_mlp2d)


def kernel(node_feats, edge_feats, edge_index, W_edge, b_edge, W_mlp, b_mlp):
    eh = _edge_encoder(edge_feats, W_edge, b_edge.reshape(1, _D))
    agg_cat = _sc_aggregate(node_feats, eh, edge_index)
    return _output_mlp(node_feats, agg_cat, W_mlp, b_mlp.reshape(1, _D))
